# Initial kernel scaffold; baseline (speedup 1.0000x reference)
import jax
import jax.numpy as jnp
from jax.experimental import pallas as pl


def kernel(x, W_real, W_imag):
    raise NotImplementedError("write your pallas kernel here")


# speedup vs baseline: 1.2873x; 1.2873x over previous
import functools

import jax
import jax.numpy as jnp
from jax import lax
from jax.experimental import pallas as pl
from jax.experimental.pallas import tpu as pltpu
from jax.experimental.pallas import tpu_sc as plsc

V = 1000000
D = 32
B = 4096
S = 200
N = B * S

NC = 2
NS = 16
NW = NC * NS

G = 4
CH = G * 128
PER_W = N // NW
CHUNKS = PER_W // CH

_mesh = plsc.VectorSubcoreMesh(core_axis_name="c", subcore_axis_name="s")


@functools.partial(
    pl.kernel,
    out_type=(
        jax.ShapeDtypeStruct((N, D), jnp.float32),
        jax.ShapeDtypeStruct((N, D), jnp.float32),
    ),
    mesh=_mesh,
    compiler_params=pltpu.CompilerParams(use_tc_tiling_on_sc=False),
    scratch_types=[
        pltpu.VMEM((G, 128), jnp.int32),
        pltpu.VMEM((CH, D), jnp.float32),
        pltpu.VMEM((CH, D), jnp.float32),
        pltpu.SemaphoreType.DMA,
    ],
)
def _sc_gather(xf, wr, wi, out_r, out_i, idx_v, rows_r, rows_i, sem):
    wid = lax.axis_index("s") * NC + lax.axis_index("c")
    base_irow = wid * (PER_W // 128)

    def chunk(c, carry):
        irow = base_irow + c * G
        pltpu.sync_copy(xf.at[pl.ds(irow, G)], idx_v)
        copies = []
        for g in range(G):
            copies.append(pltpu.async_copy(
                wr.at[idx_v.at[g]], rows_r.at[pl.ds(g * 128, 128)], sem))
            copies.append(pltpu.async_copy(
                wi.at[idx_v.at[g]], rows_i.at[pl.ds(g * 128, 128)], sem))
        for cp in copies:
            cp.wait()
        obase = irow * 128
        pltpu.sync_copy(rows_r, out_r.at[pl.ds(obase, CH)])
        pltpu.sync_copy(rows_i, out_i.at[pl.ds(obase, CH)])
        return carry

    lax.fori_loop(0, CHUNKS, chunk, 0)


def kernel(x, W_real, W_imag):
    xf = x.reshape(N // 128, 128)
    r, i = _sc_gather(xf, W_real, W_imag)
    return lax.complex(r.reshape(B, S, D), i.reshape(B, S, D))
